# Initial kernel scaffold; baseline (speedup 1.0000x reference)
import jax
import jax.numpy as jnp
from jax.experimental import pallas as pl


def kernel(X_snapshots, edge_index, dt_history, W_in, b_in, W_prop, b_prop, Wt1, bt1, Wt2, bt2, W1, b1, W2, b2):
    raise NotImplementedError("write your pallas kernel here")


# speedup vs baseline: 6.4537x; 6.4537x over previous
import dataclasses
import functools

import jax
import jax.numpy as jnp
from jax import lax
from jax.experimental import pallas as pl
from jax.experimental.pallas import tpu as pltpu
from jax.experimental.pallas import tpu_sc as plsc

N = 10000
FE = 128
NC = 2
NS = 16
NW = NC * NS
L = 16
CH = 128
CPT = 79
EPT = CH * CPT
E_PAD = EPT * NW
N_PAD = 10112
RPT = N_PAD // NS

_mesh = plsc.VectorSubcoreMesh(core_axis_name="c", subcore_axis_name="s")

_sc_params = pltpu.CompilerParams()
if "needs_layout_passes" in pltpu.CompilerParams.__dataclass_fields__:
    _sc_params = dataclasses.replace(_sc_params, needs_layout_passes=False)


@functools.partial(
    pl.kernel,
    out_type=jax.ShapeDtypeStruct((NW, N_PAD), jnp.float32),
    mesh=_mesh,
    scratch_types=[
        pltpu.VMEM((N_PAD,), jnp.float32),
        pltpu.VMEM((EPT,), jnp.int32),
    ],
    compiler_params=_sc_params,
)
def _sc_degree(dst_hbm, out_hbm, acc_v, idx_v):
    c = lax.axis_index("c")
    s = lax.axis_index("s")
    wid = s * NC + c
    z16 = jnp.zeros((L,), jnp.float32)

    @pl.loop(0, N_PAD, step=L)
    def _(i):
        acc_v[pl.ds(i, L)] = z16

    pltpu.sync_copy(dst_hbm.at[pl.ds(wid * EPT, EPT)], idx_v)
    ones = jnp.ones((L,), jnp.float32)

    @pl.loop(0, EPT, step=L)
    def _(i):
        plsc.addupdate_scatter(acc_v, [idx_v[pl.ds(i, L)]], ones)

    pltpu.sync_copy(acc_v, out_hbm.at[wid])


@functools.partial(
    pl.kernel,
    out_type=jax.ShapeDtypeStruct((NC, N_PAD, FE), jnp.float32),
    mesh=_mesh,
    scratch_types=[
        pltpu.VMEM((CH,), jnp.int32),
        pltpu.VMEM((CH,), jnp.int32),
        pltpu.VMEM((CH, FE), jnp.float32),
        pltpu.VMEM((CPT, FE), jnp.float32),
        pltpu.VMEM_SHARED((N_PAD, FE), jnp.float32),
        pltpu.SemaphoreType.DMA,
    ],
)
def _sc_round(src_hbm, dst_hbm, g_hbm, out_hbm, sidx, didx, rows, zbuf,
              acc_sh, sem):
    c = lax.axis_index("c")
    s = lax.axis_index("s")
    wid = s * NC + c
    z16 = jnp.zeros((L,), jnp.float32)

    @pl.loop(0, CPT)
    def _(r):
        @pl.loop(0, FE, step=L)
        def _(j):
            zbuf[r, pl.ds(j, L)] = z16

    for k in range(N_PAD // (NS * CPT)):
        pltpu.sync_copy(zbuf, acc_sh.at[pl.ds(s * RPT + k * CPT, CPT)])
    plsc.subcore_barrier()

    @pl.loop(0, CPT)
    def _(ch):
        off = wid * EPT + ch * CH
        pltpu.sync_copy(src_hbm.at[pl.ds(off, CH)], sidx)
        pltpu.sync_copy(dst_hbm.at[pl.ds(off, CH)], didx)
        pltpu.async_copy(g_hbm.at[sidx], rows, sem).wait()
        pltpu.sync_copy(rows, acc_sh.at[didx], add=True)

    plsc.subcore_barrier()
    pltpu.sync_copy(acc_sh.at[pl.ds(s * RPT, RPT)],
                    out_hbm.at[c, pl.ds(s * RPT, RPT)])


def _tc_proj_body(deg_ref, x_ref, win_ref, bin_ref, isd_ref, g_ref):
    deg = jnp.sum(deg_ref[...][:, :N], axis=0)
    isd = lax.rsqrt(deg + 1.0)
    isd_ref[...] = isd[:, None]
    h = jnp.maximum(
        jnp.dot(x_ref[...], win_ref[...], preferred_element_type=jnp.float32)
        + bin_ref[...], 0.0)
    g_ref[...] = h * isd[:, None]


def _tc_round_body(s_ref, isd_ref, w_ref, b_ref, g_ref):
    isd = isd_ref[...]
    agg = (s_ref[0, :N, :] + s_ref[1, :N, :]) * isd
    h = jnp.maximum(
        jnp.dot(agg, w_ref[...], preferred_element_type=jnp.float32)
        + b_ref[...], 0.0)
    g_ref[...] = h * isd


def _tc_final_body(s_ref, isd_ref, wp_ref, bp_ref, dt_ref, wt1_ref, bt1_ref,
                   wt2_ref, bt2_ref, w1a_ref, w1b_ref, b1_ref, w2_ref, b2_ref,
                   mu_ref, ls_ref, lam_ref, h_ref):
    isd = isd_ref[...]
    agg = (s_ref[0, :N, :] + s_ref[1, :N, :]) * isd
    hl = jnp.maximum(
        jnp.dot(agg, wp_ref[...], preferred_element_type=jnp.float32)
        + bp_ref[...], 0.0)
    h_ref[...] = hl
    e = jnp.maximum(dt_ref[...] * wt1_ref[...] + bt1_ref[...], 0.0)
    me = jnp.mean(e, axis=0, keepdims=True)
    ht = jnp.tanh(
        jnp.dot(me, wt2_ref[...], preferred_element_type=jnp.float32)
        + bt2_ref[...])
    const = jnp.dot(ht, w1b_ref[...], preferred_element_type=jnp.float32) \
        + b1_ref[...]
    hidden = jnp.maximum(
        jnp.dot(hl, w1a_ref[...], preferred_element_type=jnp.float32)
        + const, 0.0)
    out = jnp.dot(hidden, w2_ref[...], preferred_element_type=jnp.float32) \
        + b2_ref[...]
    mu = out[:, 0:1]
    ls2 = out[:, 1:2]
    mu_ref[...] = mu
    ls_ref[...] = ls2
    lam_ref[...] = jnp.exp(mu + 0.5 * jnp.exp(2.0 * ls2))


_tc_proj = pl.pallas_call(
    _tc_proj_body,
    out_shape=[
        jax.ShapeDtypeStruct((N, 1), jnp.float32),
        jax.ShapeDtypeStruct((N, FE), jnp.float32),
    ],
)

_tc_round = pl.pallas_call(
    _tc_round_body,
    out_shape=jax.ShapeDtypeStruct((N, FE), jnp.float32),
)

_tc_final = pl.pallas_call(
    _tc_final_body,
    out_shape=[
        jax.ShapeDtypeStruct((N, 1), jnp.float32),
        jax.ShapeDtypeStruct((N, 1), jnp.float32),
        jax.ShapeDtypeStruct((N, 1), jnp.float32),
        jax.ShapeDtypeStruct((N, FE), jnp.float32),
    ],
)


def kernel(X_snapshots, edge_index, dt_history, W_in, b_in, W_prop, b_prop,
           Wt1, bt1, Wt2, bt2, W1, b1, W2, b2):
    X = X_snapshots[-1]
    src = edge_index[0].astype(jnp.int32)
    dst = edge_index[1].astype(jnp.int32)
    npad = E_PAD - src.shape[0]
    src_pad = jnp.concatenate([src, jnp.zeros((npad,), jnp.int32)])
    dst_pad = jnp.concatenate([dst, jnp.full((npad,), N, jnp.int32)])

    deg_parts = _sc_degree(dst_pad)
    isd, G = _tc_proj(deg_parts, X, W_in, b_in.reshape(1, FE))

    for _ in range(2):
        S = _sc_round(src_pad, dst_pad, G)
        G = _tc_round(S, isd, W_prop, b_prop.reshape(1, FE))
    S = _sc_round(src_pad, dst_pad, G)

    w2p = jnp.pad(W2, ((0, 0), (0, FE - W2.shape[1])))
    b2p = jnp.pad(b2, (0, FE - b2.shape[0])).reshape(1, FE)
    mu, ls, lam, h_last = _tc_final(
        S, isd, W_prop, b_prop.reshape(1, FE),
        dt_history.reshape(-1, 1), Wt1, bt1.reshape(1, -1), Wt2,
        bt2.reshape(1, -1), W1[:FE, :], W1[FE:, :], b1.reshape(1, -1),
        w2p, b2p)
    return mu[:, 0], ls[:, 0], lam[:, 0], h_last
